# Initial kernel scaffold; baseline (speedup 1.0000x reference)
import jax
import jax.numpy as jnp
from jax.experimental import pallas as pl


def kernel(nodes, grid, edge_index, edge_attr, batch_size, image_size, params):
    raise NotImplementedError("write your pallas kernel here")


# speedup vs baseline: 3.1158x; 3.1158x over previous
import functools

import jax
import jax.numpy as jnp
from jax import lax
from jax.experimental import pallas as pl
from jax.experimental.pallas import tpu as pltpu
from jax.experimental.pallas import tpu_sc as plsc

N = 10000
E = 160000
LAT = 32
KD = 64
EDIM = 5

CHUNK = 128
EPAD = 163840
NW = 32
WCH = EPAD // CHUNK // NW
KF = 20
NPH = WCH // KF
AGG_ROWS = 10240
RPT = AGG_ROWS // 16
PAD_DST = N

@functools.cache
def _sc_mesh():
    return plsc.VectorSubcoreMesh(core_axis_name="c", subcore_axis_name="s")


_SC_PARAMS = pltpu.CompilerParams(use_tc_tiling_on_sc=False)


def _sc_gather(x, srcp):
    @functools.partial(
        pl.kernel,
        out_type=jax.ShapeDtypeStruct((EPAD, LAT), jnp.float32),
        mesh=_sc_mesh(),
        compiler_params=_SC_PARAMS,
        scratch_types=[
            pltpu.VMEM((KF, CHUNK), jnp.int32),
            pltpu.VMEM((KF * CHUNK, LAT), jnp.float32),
            pltpu.SemaphoreType.DMA,
            pltpu.SemaphoreType.DMA,
        ],
    )
    def k(x_hbm, src_hbm, xs_hbm, idx_v, rows_v, semi, semg):
        wid = lax.axis_index("s") * 2 + lax.axis_index("c")
        cb = wid * WCH
        for s in range(NPH):
            base = (cb + s * KF) * CHUNK
            hs = [
                pltpu.async_copy(
                    src_hbm.at[pl.ds(base + t * CHUNK, CHUNK)], idx_v.at[t], semi
                )
                for t in range(KF)
            ]
            for h in hs:
                h.wait()
            gs = [
                pltpu.async_copy(
                    x_hbm.at[idx_v.at[t]],
                    rows_v.at[pl.ds(t * CHUNK, CHUNK)],
                    semg,
                )
                for t in range(KF)
            ]
            for h in gs:
                h.wait()
            pltpu.sync_copy(rows_v, xs_hbm.at[pl.ds(base, KF * CHUNK)])

    return k(x, srcp)


def _sc_scatter(msgp, dstp, zeros_tile):
    @functools.partial(
        pl.kernel,
        out_type=jax.ShapeDtypeStruct((2, AGG_ROWS, LAT), jnp.float32),
        mesh=_sc_mesh(),
        compiler_params=_SC_PARAMS,
        scratch_types=[
            pltpu.VMEM((KF, CHUNK), jnp.int32),
            pltpu.VMEM((KF * CHUNK, LAT), jnp.float32),
            pltpu.VMEM_SHARED((AGG_ROWS, LAT), jnp.float32),
            pltpu.SemaphoreType.DMA,
            pltpu.SemaphoreType.DMA,
            pltpu.SemaphoreType.DMA,
        ],
    )
    def k(msg_hbm, dst_hbm, z_hbm, out_hbm, idx_v, msg_v, shared, semi, semm, sems):
        cid = lax.axis_index("c")
        sid = lax.axis_index("s")
        wid = sid * 2 + cid

        if True:
            pltpu.sync_copy(z_hbm, shared.at[pl.ds(sid * RPT, RPT)])
            plsc.subcore_barrier()
            cb = wid * WCH
            pend = []
            for s in range(NPH):
                base = (cb + s * KF) * CHUNK
                for h in pend:
                    h.wait()
                pend = []
                hs = [
                    pltpu.async_copy(
                        dst_hbm.at[pl.ds(base + t * CHUNK, CHUNK)], idx_v.at[t], semi
                    )
                    for t in range(KF)
                ]
                hm = pltpu.async_copy(
                    msg_hbm.at[pl.ds(base, KF * CHUNK)], msg_v, semm
                )
                for h in hs:
                    h.wait()
                hm.wait()
                pend = [
                    pltpu.async_copy(
                        msg_v.at[pl.ds(t * CHUNK, CHUNK)],
                        shared.at[idx_v.at[t]],
                        sems,
                        add=True,
                    )
                    for t in range(KF)
                ]
            for h in pend:
                h.wait()
            plsc.subcore_barrier()
            pltpu.sync_copy(
                shared.at[pl.ds(sid * RPT, RPT)],
                out_hbm.at[cid, pl.ds(sid * RPT, RPT)],
            )

    return k(msgp, dstp, zeros_tile)


def _edge_msgs(edge_attr, xsp, bp):
    ET = 1280
    steps = E // ET
    wk1 = bp["Wk1"]
    bk1 = bp["bk1"].reshape(1, KD)
    wk2 = bp["Wk2"]
    bk2 = bp["bk2"].reshape(1, LAT * LAT)
    pmat = (
        jnp.arange(LAT * LAT, dtype=jnp.int32)[:, None] // LAT
        == jnp.arange(LAT, dtype=jnp.int32)[None, :]
    ).astype(jnp.float32)

    def body(ea_ref, xs_ref, w1_ref, b1_ref, w2_ref, b2_ref, p_ref, msg_ref):
        h = jax.nn.gelu(
            jnp.dot(ea_ref[...], w1_ref[...], preferred_element_type=jnp.float32)
            + b1_ref[...]
        )
        kmat = (
            jnp.dot(h, w2_ref[...], preferred_element_type=jnp.float32) + b2_ref[...]
        )
        xsr = jnp.concatenate([xs_ref[...]] * LAT, axis=1)
        msg_ref[...] = jnp.dot(
            kmat * xsr, p_ref[...], preferred_element_type=jnp.float32
        )

    return pl.pallas_call(
        body,
        grid=(steps,),
        in_specs=[
            pl.BlockSpec((ET, EDIM), lambda i: (i, 0)),
            pl.BlockSpec((ET, LAT), lambda i: (i, 0)),
            pl.BlockSpec((EDIM, KD), lambda i: (0, 0)),
            pl.BlockSpec((1, KD), lambda i: (0, 0)),
            pl.BlockSpec((KD, LAT * LAT), lambda i: (0, 0)),
            pl.BlockSpec((1, LAT * LAT), lambda i: (0, 0)),
            pl.BlockSpec((LAT * LAT, LAT), lambda i: (0, 0)),
        ],
        out_specs=pl.BlockSpec((ET, LAT), lambda i: (i, 0)),
        out_shape=jax.ShapeDtypeStruct((EPAD, LAT), jnp.float32),
    )(edge_attr, xsp, wk1, bk1, wk2, bk2, pmat)


def _encode(xin, dparts, pp):
    def body(xi_ref, dp_ref, w1_ref, b1_ref, w2_ref, b2_ref, x0_ref, rd_ref):
        h = jax.nn.gelu(
            jnp.dot(xi_ref[...], w1_ref[...], preferred_element_type=jnp.float32)
            + b1_ref[...]
        )
        x0_ref[...] = jax.nn.gelu(
            jnp.dot(h, w2_ref[...], preferred_element_type=jnp.float32) + b2_ref[...]
        )
        deg = dp_ref[0, :N, 0:1] + dp_ref[1, :N, 0:1]
        rd = 1.0 / jnp.maximum(deg, 1.0)
        rd_ref[...] = jnp.broadcast_to(rd, (N, LAT))

    return pl.pallas_call(
        body,
        out_shape=(
            jax.ShapeDtypeStruct((N, LAT), jnp.float32),
            jax.ShapeDtypeStruct((N, LAT), jnp.float32),
        ),
    )(
        xin,
        dparts,
        pp["W1"],
        pp["b1"].reshape(1, -1),
        pp["W2"],
        pp["b2"].reshape(1, -1),
    )


def _update(x, parts, rdeg, wr, br, apply_gelu):
    def body(x_ref, p_ref, rd_ref, wr_ref, br_ref, o_ref):
        agg = (p_ref[0, :N, :] + p_ref[1, :N, :]) * rd_ref[...]
        y = (
            jnp.dot(x_ref[...], wr_ref[...], preferred_element_type=jnp.float32)
            + br_ref[...]
            + agg
        )
        if apply_gelu:
            y = jax.nn.gelu(y)
        o_ref[...] = y

    return pl.pallas_call(
        body,
        out_shape=jax.ShapeDtypeStruct((N, LAT), jnp.float32),
    )(x, parts, rdeg, wr, br.reshape(1, LAT))


def _decode(x, dp):
    def body(x_ref, w1_ref, b1_ref, w2r_ref, b2_ref, o_ref):
        h = jax.nn.gelu(
            jnp.dot(x_ref[...], w1_ref[...], preferred_element_type=jnp.float32)
            + b1_ref[...]
        )
        o_ref[...] = (
            jnp.sum(h * w2r_ref[...], axis=1, keepdims=True) + b2_ref[...]
        )

    return pl.pallas_call(
        body,
        out_shape=jax.ShapeDtypeStruct((N, 1), jnp.float32),
    )(
        x,
        dp["W1"],
        dp["b1"].reshape(1, -1),
        dp["W2"].reshape(1, -1),
        dp["b2"].reshape(1, 1),
    )


def kernel(nodes, grid, edge_index, edge_attr, batch_size, image_size, params):
    src = edge_index[0].astype(jnp.int32)
    dst = edge_index[1].astype(jnp.int32)
    srcp = jnp.concatenate([src, jnp.zeros((EPAD - E,), jnp.int32)])
    dstp = jnp.concatenate([dst, jnp.full((EPAD - E,), PAD_DST, jnp.int32)])
    zeros_tile = jnp.zeros((RPT, LAT), jnp.float32)
    ones_msg = jnp.ones((EPAD, LAT), jnp.float32)

    dparts = _sc_scatter(ones_msg, dstp, zeros_tile)
    xin = jnp.concatenate([nodes, grid], axis=1)
    x, rdeg = _encode(xin, dparts, params["projector"])

    for b in range(2):
        bp = params["blocks"][b]
        for p in range(2):
            xs = _sc_gather(x, srcp)
            msg = _edge_msgs(edge_attr, xs, bp)
            parts = _sc_scatter(msg, dstp, zeros_tile)
            x = _update(x, parts, rdeg, bp["Wr"], bp["br"], not (b == 1 and p == 1))

    return _decode(x, params["decoder"])


# speedup vs baseline: 3.1646x; 1.0157x over previous
import functools

import jax
import jax.numpy as jnp
from jax import lax
from jax.experimental import pallas as pl
from jax.experimental.pallas import tpu as pltpu
from jax.experimental.pallas import tpu_sc as plsc

N = 10000
E = 160000
LAT = 32
KD = 64
EDIM = 5

CHUNK = 128
EPAD = 163840
NW = 32
WCH = EPAD // CHUNK // NW
KF = 20
NPH = WCH // KF
AGG_ROWS = 10240
RPT = AGG_ROWS // 16
PAD_DST = N

@functools.cache
def _sc_mesh():
    return plsc.VectorSubcoreMesh(core_axis_name="c", subcore_axis_name="s")


_SC_PARAMS = pltpu.CompilerParams(use_tc_tiling_on_sc=False)


def _sc_gather(x, srcp):
    @functools.partial(
        pl.kernel,
        out_type=jax.ShapeDtypeStruct((EPAD, LAT), jnp.float32),
        mesh=_sc_mesh(),
        compiler_params=_SC_PARAMS,
        scratch_types=[
            pltpu.VMEM((KF, CHUNK), jnp.int32),
            pltpu.VMEM((KF * CHUNK, LAT), jnp.float32),
            pltpu.SemaphoreType.DMA,
            pltpu.SemaphoreType.DMA,
        ],
    )
    def k(x_hbm, src_hbm, xs_hbm, idx_v, rows_v, semi, semg):
        wid = lax.axis_index("s") * 2 + lax.axis_index("c")
        cb = wid * WCH
        for s in range(NPH):
            base = (cb + s * KF) * CHUNK
            hs = [
                pltpu.async_copy(
                    src_hbm.at[pl.ds(base + t * CHUNK, CHUNK)], idx_v.at[t], semi
                )
                for t in range(KF)
            ]
            for h in hs:
                h.wait()
            gs = [
                pltpu.async_copy(
                    x_hbm.at[idx_v.at[t]],
                    rows_v.at[pl.ds(t * CHUNK, CHUNK)],
                    semg,
                )
                for t in range(KF)
            ]
            for h in gs:
                h.wait()
            pltpu.sync_copy(rows_v, xs_hbm.at[pl.ds(base, KF * CHUNK)])

    return k(x, srcp)


def _sc_scatter(msgp, dstp, zeros_tile):
    @functools.partial(
        pl.kernel,
        out_type=jax.ShapeDtypeStruct((2, AGG_ROWS, LAT), jnp.float32),
        mesh=_sc_mesh(),
        compiler_params=_SC_PARAMS,
        scratch_types=[
            pltpu.VMEM((KF, CHUNK), jnp.int32),
            pltpu.VMEM((KF * CHUNK, LAT), jnp.float32),
            pltpu.VMEM_SHARED((AGG_ROWS, LAT), jnp.float32),
            pltpu.SemaphoreType.DMA,
            pltpu.SemaphoreType.DMA,
            pltpu.SemaphoreType.DMA,
        ],
    )
    def k(msg_hbm, dst_hbm, z_hbm, out_hbm, idx_v, msg_v, shared, semi, semm, sems):
        cid = lax.axis_index("c")
        sid = lax.axis_index("s")
        wid = sid * 2 + cid

        if True:
            pltpu.sync_copy(z_hbm, shared.at[pl.ds(sid * RPT, RPT)])
            plsc.subcore_barrier()
            cb = wid * WCH
            pend = []
            for s in range(NPH):
                base = (cb + s * KF) * CHUNK
                for h in pend:
                    h.wait()
                pend = []
                hs = [
                    pltpu.async_copy(
                        dst_hbm.at[pl.ds(base + t * CHUNK, CHUNK)], idx_v.at[t], semi
                    )
                    for t in range(KF)
                ]
                hm = pltpu.async_copy(
                    msg_hbm.at[pl.ds(base, KF * CHUNK)], msg_v, semm
                )
                for h in hs:
                    h.wait()
                hm.wait()
                pend = [
                    pltpu.async_copy(
                        msg_v.at[pl.ds(t * CHUNK, CHUNK)],
                        shared.at[idx_v.at[t]],
                        sems,
                        add=True,
                    )
                    for t in range(KF)
                ]
            for h in pend:
                h.wait()
            plsc.subcore_barrier()
            pltpu.sync_copy(
                shared.at[pl.ds(sid * RPT, RPT)],
                out_hbm.at[cid, pl.ds(sid * RPT, RPT)],
            )

    return k(msgp, dstp, zeros_tile)


def _sc_deg(dstp, zeros_tile, ones_small):
    @functools.partial(
        pl.kernel,
        out_type=jax.ShapeDtypeStruct((2, AGG_ROWS, LAT), jnp.float32),
        mesh=_sc_mesh(),
        compiler_params=_SC_PARAMS,
        scratch_types=[
            pltpu.VMEM((KF, CHUNK), jnp.int32),
            pltpu.VMEM((KF * CHUNK, LAT), jnp.float32),
            pltpu.VMEM_SHARED((AGG_ROWS, LAT), jnp.float32),
            pltpu.SemaphoreType.DMA,
            pltpu.SemaphoreType.DMA,
        ],
    )
    def k(dst_hbm, z_hbm, ones_hbm, out_hbm, idx_v, msg_v, shared, semi, sems):
        cid = lax.axis_index("c")
        sid = lax.axis_index("s")
        wid = sid * 2 + cid
        pltpu.sync_copy(z_hbm, shared.at[pl.ds(sid * RPT, RPT)])
        pltpu.sync_copy(ones_hbm, msg_v)
        plsc.subcore_barrier()
        cb = wid * WCH
        pend = []
        for s in range(NPH):
            base = (cb + s * KF) * CHUNK
            for h in pend:
                h.wait()
            pend = []
            hs = [
                pltpu.async_copy(
                    dst_hbm.at[pl.ds(base + t * CHUNK, CHUNK)], idx_v.at[t], semi
                )
                for t in range(KF)
            ]
            for h in hs:
                h.wait()
            pend = [
                pltpu.async_copy(
                    msg_v.at[pl.ds(t * CHUNK, CHUNK)],
                    shared.at[idx_v.at[t]],
                    sems,
                    add=True,
                )
                for t in range(KF)
            ]
        for h in pend:
            h.wait()
        plsc.subcore_barrier()
        pltpu.sync_copy(
            shared.at[pl.ds(sid * RPT, RPT)],
            out_hbm.at[cid, pl.ds(sid * RPT, RPT)],
        )

    return k(dstp, zeros_tile, ones_small)


def _edge_msgs(edge_attr, xsp, bp):
    ET = 1280
    steps = E // ET
    wk1 = bp["Wk1"]
    bk1 = bp["bk1"].reshape(1, KD)
    wk2 = bp["Wk2"]
    bk2 = bp["bk2"].reshape(1, LAT * LAT)
    pmat = (
        jnp.arange(LAT * LAT, dtype=jnp.int32)[:, None] // LAT
        == jnp.arange(LAT, dtype=jnp.int32)[None, :]
    ).astype(jnp.float32)

    def body(ea_ref, xs_ref, w1_ref, b1_ref, w2_ref, b2_ref, p_ref, msg_ref):
        h = jax.nn.gelu(
            jnp.dot(ea_ref[...], w1_ref[...], preferred_element_type=jnp.float32)
            + b1_ref[...]
        )
        kmat = (
            jnp.dot(h, w2_ref[...], preferred_element_type=jnp.float32) + b2_ref[...]
        )
        xsr = jnp.concatenate([xs_ref[...]] * LAT, axis=1)
        msg_ref[...] = jnp.dot(
            kmat * xsr, p_ref[...], preferred_element_type=jnp.float32
        )

    return pl.pallas_call(
        body,
        grid=(steps,),
        in_specs=[
            pl.BlockSpec((ET, EDIM), lambda i: (i, 0)),
            pl.BlockSpec((ET, LAT), lambda i: (i, 0)),
            pl.BlockSpec((EDIM, KD), lambda i: (0, 0)),
            pl.BlockSpec((1, KD), lambda i: (0, 0)),
            pl.BlockSpec((KD, LAT * LAT), lambda i: (0, 0)),
            pl.BlockSpec((1, LAT * LAT), lambda i: (0, 0)),
            pl.BlockSpec((LAT * LAT, LAT), lambda i: (0, 0)),
        ],
        out_specs=pl.BlockSpec((ET, LAT), lambda i: (i, 0)),
        out_shape=jax.ShapeDtypeStruct((EPAD, LAT), jnp.float32),
    )(edge_attr, xsp, wk1, bk1, wk2, bk2, pmat)


def _encode(xin, pp):
    def body(xi_ref, w1_ref, b1_ref, w2_ref, b2_ref, x0_ref):
        h = jax.nn.gelu(
            jnp.dot(xi_ref[...], w1_ref[...], preferred_element_type=jnp.float32)
            + b1_ref[...]
        )
        x0_ref[...] = jax.nn.gelu(
            jnp.dot(h, w2_ref[...], preferred_element_type=jnp.float32) + b2_ref[...]
        )

    return pl.pallas_call(
        body,
        out_shape=jax.ShapeDtypeStruct((N, LAT), jnp.float32),
    )(
        xin,
        pp["W1"],
        pp["b1"].reshape(1, -1),
        pp["W2"],
        pp["b2"].reshape(1, -1),
    )


def _agg(p_ref, dp_ref):
    deg = dp_ref[0, :N, 0:1] + dp_ref[1, :N, 0:1]
    rd = 1.0 / jnp.maximum(deg, 1.0)
    return (p_ref[0, :N, :] + p_ref[1, :N, :]) * rd


def _update(x, parts, dparts, wr, br, apply_gelu):
    def body(x_ref, p_ref, dp_ref, wr_ref, br_ref, o_ref):
        y = (
            jnp.dot(x_ref[...], wr_ref[...], preferred_element_type=jnp.float32)
            + br_ref[...]
            + _agg(p_ref, dp_ref)
        )
        if apply_gelu:
            y = jax.nn.gelu(y)
        o_ref[...] = y

    return pl.pallas_call(
        body,
        out_shape=jax.ShapeDtypeStruct((N, LAT), jnp.float32),
    )(x, parts, dparts, wr, br.reshape(1, LAT))


def _update_decode(x, parts, dparts, wr, br, dp):
    def body(x_ref, p_ref, dp_ref, wr_ref, br_ref, w1_ref, b1_ref, w2r_ref,
             b2_ref, o_ref):
        y = (
            jnp.dot(x_ref[...], wr_ref[...], preferred_element_type=jnp.float32)
            + br_ref[...]
            + _agg(p_ref, dp_ref)
        )
        h = jax.nn.gelu(
            jnp.dot(y, w1_ref[...], preferred_element_type=jnp.float32)
            + b1_ref[...]
        )
        o_ref[...] = jnp.sum(h * w2r_ref[...], axis=1, keepdims=True) + b2_ref[...]

    return pl.pallas_call(
        body,
        out_shape=jax.ShapeDtypeStruct((N, 1), jnp.float32),
    )(
        x,
        parts,
        dparts,
        wr,
        br.reshape(1, LAT),
        dp["W1"],
        dp["b1"].reshape(1, -1),
        dp["W2"].reshape(1, -1),
        dp["b2"].reshape(1, 1),
    )


def kernel(nodes, grid, edge_index, edge_attr, batch_size, image_size, params):
    src = edge_index[0].astype(jnp.int32)
    dst = edge_index[1].astype(jnp.int32)
    srcp = jnp.concatenate([src, jnp.zeros((EPAD - E,), jnp.int32)])
    dstp = jnp.concatenate([dst, jnp.full((EPAD - E,), PAD_DST, jnp.int32)])
    zeros_tile = jnp.zeros((RPT, LAT), jnp.float32)
    ones_small = jnp.ones((KF * CHUNK, LAT), jnp.float32)

    dparts = _sc_deg(dstp, zeros_tile, ones_small)
    xin = jnp.concatenate([nodes, grid], axis=1)
    x = _encode(xin, params["projector"])

    for b in range(2):
        bp = params["blocks"][b]
        for p in range(2):
            xs = _sc_gather(x, srcp)
            msg = _edge_msgs(edge_attr, xs, bp)
            parts = _sc_scatter(msg, dstp, zeros_tile)
            if b == 1 and p == 1:
                return _update_decode(
                    x, parts, dparts, bp["Wr"], bp["br"], params["decoder"]
                )
            x = _update(x, parts, dparts, bp["Wr"], bp["br"], True)


# speedup vs baseline: 4.1514x; 1.3118x over previous
import functools

import jax
import jax.numpy as jnp
import numpy as np
from jax import lax
from jax.experimental import pallas as pl
from jax.experimental.pallas import tpu as pltpu
from jax.experimental.pallas import tpu_sc as plsc

N = 10000
E = 160000
LAT = 32
KD = 64
EDIM = 5

CHUNK = 128
EPAD = 163840
NW = 32
WCH = EPAD // CHUNK // NW
KF = 20
NPH = WCH // KF
AGG_ROWS = 10240
RPT = AGG_ROWS // 16
PAD_DST = N
PK = 4
N4 = AGG_ROWS // PK

@functools.cache
def _sc_mesh():
    return plsc.VectorSubcoreMesh(core_axis_name="c", subcore_axis_name="s")


_SC_PARAMS = pltpu.CompilerParams(use_tc_tiling_on_sc=False)


def _sc_gather(x, srcp):
    @functools.partial(
        pl.kernel,
        out_type=jax.ShapeDtypeStruct((EPAD, LAT), jnp.float32),
        mesh=_sc_mesh(),
        compiler_params=_SC_PARAMS,
        scratch_types=[
            pltpu.VMEM((KF, CHUNK), jnp.int32),
            pltpu.VMEM((KF * CHUNK, LAT), jnp.float32),
            pltpu.SemaphoreType.DMA,
            pltpu.SemaphoreType.DMA,
        ],
    )
    def k(x_hbm, src_hbm, xs_hbm, idx_v, rows_v, semi, semg):
        wid = lax.axis_index("s") * 2 + lax.axis_index("c")
        cb = wid * WCH
        for s in range(NPH):
            base = (cb + s * KF) * CHUNK
            hs = [
                pltpu.async_copy(
                    src_hbm.at[pl.ds(base + t * CHUNK, CHUNK)], idx_v.at[t], semi
                )
                for t in range(KF)
            ]
            for h in hs:
                h.wait()
            gs = [
                pltpu.async_copy(
                    x_hbm.at[idx_v.at[t]],
                    rows_v.at[pl.ds(t * CHUNK, CHUNK)],
                    semg,
                )
                for t in range(KF)
            ]
            for h in gs:
                h.wait()
            pltpu.sync_copy(rows_v, xs_hbm.at[pl.ds(base, KF * CHUNK)])

    return k(x, srcp)


def _sc_scatter(msgp, dstp, zeros_tile):
    @functools.partial(
        pl.kernel,
        out_type=jax.ShapeDtypeStruct((2, AGG_ROWS, LAT), jnp.float32),
        mesh=_sc_mesh(),
        compiler_params=_SC_PARAMS,
        scratch_types=[
            pltpu.VMEM((KF, CHUNK), jnp.int32),
            pltpu.VMEM((KF * CHUNK, LAT), jnp.float32),
            pltpu.VMEM_SHARED((AGG_ROWS, LAT), jnp.float32),
            pltpu.SemaphoreType.DMA,
            pltpu.SemaphoreType.DMA,
            pltpu.SemaphoreType.DMA,
        ],
    )
    def k(msg_hbm, dst_hbm, z_hbm, out_hbm, idx_v, msg_v, shared, semi, semm, sems):
        cid = lax.axis_index("c")
        sid = lax.axis_index("s")
        wid = sid * 2 + cid

        if True:
            pltpu.sync_copy(z_hbm, shared.at[pl.ds(sid * RPT, RPT)])
            plsc.subcore_barrier()
            cb = wid * WCH
            pend = []
            for s in range(NPH):
                base = (cb + s * KF) * CHUNK
                for h in pend:
                    h.wait()
                pend = []
                hs = [
                    pltpu.async_copy(
                        dst_hbm.at[pl.ds(base + t * CHUNK, CHUNK)], idx_v.at[t], semi
                    )
                    for t in range(KF)
                ]
                hm = pltpu.async_copy(
                    msg_hbm.at[pl.ds(base, KF * CHUNK)], msg_v, semm
                )
                for h in hs:
                    h.wait()
                hm.wait()
                pend = [
                    pltpu.async_copy(
                        msg_v.at[pl.ds(t * CHUNK, CHUNK)],
                        shared.at[idx_v.at[t]],
                        sems,
                        add=True,
                    )
                    for t in range(KF)
                ]
            for h in pend:
                h.wait()
            plsc.subcore_barrier()
            pltpu.sync_copy(
                shared.at[pl.ds(sid * RPT, RPT)],
                out_hbm.at[cid, pl.ds(sid * RPT, RPT)],
            )

    return k(msgp, dstp, zeros_tile)


def _sc_deg(dstp, zeros_tile, ones_small):
    @functools.partial(
        pl.kernel,
        out_type=jax.ShapeDtypeStruct((2, AGG_ROWS, LAT), jnp.float32),
        mesh=_sc_mesh(),
        compiler_params=_SC_PARAMS,
        scratch_types=[
            pltpu.VMEM((KF, CHUNK), jnp.int32),
            pltpu.VMEM((KF * CHUNK, LAT), jnp.float32),
            pltpu.VMEM_SHARED((AGG_ROWS, LAT), jnp.float32),
            pltpu.SemaphoreType.DMA,
            pltpu.SemaphoreType.DMA,
        ],
    )
    def k(dst_hbm, z_hbm, ones_hbm, out_hbm, idx_v, msg_v, shared, semi, sems):
        cid = lax.axis_index("c")
        sid = lax.axis_index("s")
        wid = sid * 2 + cid
        pltpu.sync_copy(z_hbm, shared.at[pl.ds(sid * RPT, RPT)])
        pltpu.sync_copy(ones_hbm, msg_v)
        plsc.subcore_barrier()
        cb = wid * WCH
        pend = []
        for s in range(NPH):
            base = (cb + s * KF) * CHUNK
            for h in pend:
                h.wait()
            pend = []
            hs = [
                pltpu.async_copy(
                    dst_hbm.at[pl.ds(base + t * CHUNK, CHUNK)], idx_v.at[t], semi
                )
                for t in range(KF)
            ]
            for h in hs:
                h.wait()
            pend = [
                pltpu.async_copy(
                    msg_v.at[pl.ds(t * CHUNK, CHUNK)],
                    shared.at[idx_v.at[t]],
                    sems,
                    add=True,
                )
                for t in range(KF)
            ]
        for h in pend:
            h.wait()
        plsc.subcore_barrier()
        pltpu.sync_copy(
            shared.at[pl.ds(sid * RPT, RPT)],
            out_hbm.at[cid, pl.ds(sid * RPT, RPT)],
        )

    return k(dstp, zeros_tile, ones_small)


def _edge_msgs(edge_attr, xsp, bp):
    ET = 1280
    steps = E // ET
    wk1 = bp["Wk1"]
    bk1 = bp["bk1"].reshape(1, KD)
    wk2 = bp["Wk2"]
    bk2 = bp["bk2"].reshape(1, LAT * LAT)
    pmat = (
        jnp.arange(LAT * LAT, dtype=jnp.int32)[:, None] // LAT
        == jnp.arange(LAT, dtype=jnp.int32)[None, :]
    ).astype(jnp.float32)

    def body(ea_ref, xs_ref, w1_ref, b1_ref, w2_ref, b2_ref, p_ref, msg_ref):
        h = jax.nn.gelu(
            jnp.dot(ea_ref[...], w1_ref[...], preferred_element_type=jnp.float32)
            + b1_ref[...]
        )
        kmat = (
            jnp.dot(h, w2_ref[...], preferred_element_type=jnp.float32) + b2_ref[...]
        )
        xs4 = xs_ref[...]
        xs = jnp.concatenate(
            [xs4[:, LAT * q : LAT * (q + 1)] for q in range(PK)], axis=0
        )
        xsr = jnp.concatenate([xs] * LAT, axis=1)
        msg = jnp.dot(kmat * xsr, p_ref[...], preferred_element_type=jnp.float32)
        ET4 = ET // PK
        msg_ref[...] = jnp.concatenate(
            [msg[ET4 * q : ET4 * (q + 1), :] for q in range(PK)], axis=1
        )

    return pl.pallas_call(
        body,
        grid=(steps,),
        in_specs=[
            pl.BlockSpec((ET, EDIM), lambda i: (i, 0)),
            pl.BlockSpec((ET // PK, PK * LAT), lambda i: (i, 0)),
            pl.BlockSpec((EDIM, KD), lambda i: (0, 0)),
            pl.BlockSpec((1, KD), lambda i: (0, 0)),
            pl.BlockSpec((KD, LAT * LAT), lambda i: (0, 0)),
            pl.BlockSpec((1, LAT * LAT), lambda i: (0, 0)),
            pl.BlockSpec((LAT * LAT, LAT), lambda i: (0, 0)),
        ],
        out_specs=pl.BlockSpec((ET // PK, PK * LAT), lambda i: (i, 0)),
        out_shape=jax.ShapeDtypeStruct((EPAD // PK, PK * LAT), jnp.float32),
    )(edge_attr, xsp, wk1, bk1, wk2, bk2, pmat)


def _bdiag(w):
    din, dout = w.shape
    z = jnp.zeros((PK * din, PK * dout), jnp.float32)
    for i in range(PK):
        z = z.at[i * din : (i + 1) * din, i * dout : (i + 1) * dout].set(w)
    return z


def _tile_b(b):
    return jnp.concatenate([b.reshape(1, -1)] * PK, axis=1)


def _encode(xin4, pp):
    def body(xi_ref, w1_ref, b1_ref, w2_ref, b2_ref, x0_ref):
        h = jax.nn.gelu(
            jnp.dot(xi_ref[...], w1_ref[...], preferred_element_type=jnp.float32)
            + b1_ref[...]
        )
        x0_ref[...] = jax.nn.gelu(
            jnp.dot(h, w2_ref[...], preferred_element_type=jnp.float32) + b2_ref[...]
        )

    return pl.pallas_call(
        body,
        out_shape=jax.ShapeDtypeStruct((N4, PK * LAT), jnp.float32),
    )(
        xin4,
        _bdiag(pp["W1"]),
        _tile_b(pp["b1"]),
        _bdiag(pp["W2"]),
        _tile_b(pp["b2"]),
    )


def _agg4(p_ref, dp_ref):
    deg = dp_ref[0] + dp_ref[1]
    rd = 1.0 / jnp.maximum(deg, 1.0)
    return (p_ref[0] + p_ref[1]) * rd


def _update(x4, parts4, dparts4, wrd, brt, apply_gelu):
    def body(x_ref, p_ref, dp_ref, wr_ref, br_ref, o_ref):
        y = (
            jnp.dot(x_ref[...], wr_ref[...], preferred_element_type=jnp.float32)
            + br_ref[...]
            + _agg4(p_ref, dp_ref)
        )
        if apply_gelu:
            y = jax.nn.gelu(y)
        o_ref[...] = y

    return pl.pallas_call(
        body,
        out_shape=jax.ShapeDtypeStruct((N4, PK * LAT), jnp.float32),
    )(x4, parts4, dparts4, wrd, brt)


def _update_decode(x4, parts4, dparts4, wrd, brt, dp):
    w2s = _bdiag(dp["W2"])
    b2t = jnp.concatenate([dp["b2"].reshape(1, 1)] * PK, axis=1)

    def body(x_ref, p_ref, dp_ref, wr_ref, br_ref, w1_ref, b1_ref, w2s_ref,
             b2_ref, o_ref):
        y = (
            jnp.dot(x_ref[...], wr_ref[...], preferred_element_type=jnp.float32)
            + br_ref[...]
            + _agg4(p_ref, dp_ref)
        )
        h = jax.nn.gelu(
            jnp.dot(y, w1_ref[...], preferred_element_type=jnp.float32)
            + b1_ref[...]
        )
        o_ref[...] = (
            jnp.dot(h, w2s_ref[...], preferred_element_type=jnp.float32)
            + b2_ref[...]
        )

    return pl.pallas_call(
        body,
        out_shape=jax.ShapeDtypeStruct((N4, PK), jnp.float32),
    )(
        x4,
        parts4,
        dparts4,
        wrd,
        brt,
        _bdiag(dp["W1"]),
        _tile_b(dp["b1"]),
        w2s,
        b2t,
    )


def kernel(nodes, grid, edge_index, edge_attr, batch_size, image_size, params):
    src = edge_index[0].astype(jnp.int32)
    dst = edge_index[1].astype(jnp.int32)
    srcp = jnp.concatenate([src, jnp.zeros((EPAD - E,), jnp.int32)])
    dstp = jnp.concatenate([dst, jnp.full((EPAD - E,), PAD_DST, jnp.int32)])
    s_ids = np.arange(EPAD)
    blk, t = s_ids // 1280, s_ids % 1280
    jg = jnp.asarray(blk * 1280 + (t % PK) * (1280 // PK) + t // PK)
    srcp = srcp[jg]
    dstp = dstp[jg]
    zeros_tile = jnp.zeros((RPT, LAT), jnp.float32)
    ones_small = jnp.ones((KF * CHUNK, LAT), jnp.float32)

    dparts = _sc_deg(dstp, zeros_tile, ones_small)
    dparts4 = dparts.reshape(2, N4, PK * LAT)
    xin = jnp.concatenate([nodes, grid], axis=1)
    xin4 = jnp.pad(xin, ((0, AGG_ROWS - N), (0, 0))).reshape(N4, PK * 12)
    x4 = _encode(xin4, params["projector"])

    for b in range(2):
        bp = params["blocks"][b]
        wrd = _bdiag(bp["Wr"])
        brt = _tile_b(bp["br"])
        for p in range(2):
            xs = _sc_gather(x4.reshape(AGG_ROWS, LAT), srcp)
            xs4 = xs.reshape(EPAD // PK, PK * LAT)
            msg4 = _edge_msgs(edge_attr, xs4, bp)
            parts = _sc_scatter(msg4.reshape(EPAD, LAT), dstp, zeros_tile)
            parts4 = parts.reshape(2, N4, PK * LAT)
            if b == 1 and p == 1:
                out4 = _update_decode(
                    x4, parts4, dparts4, wrd, brt, params["decoder"]
                )
                return out4.reshape(AGG_ROWS, 1)[:N]
            x4 = _update(x4, parts4, dparts4, wrd, brt, True)


# speedup vs baseline: 4.4451x; 1.0707x over previous
import functools

import jax
import jax.numpy as jnp
import numpy as np
from jax import lax
from jax.experimental import pallas as pl
from jax.experimental.pallas import tpu as pltpu
from jax.experimental.pallas import tpu_sc as plsc

N = 10000
E = 160000
LAT = 32
KD = 64
EDIM = 5

CHUNK = 128
EPAD = 163840
NW = 32
WCH = EPAD // CHUNK // NW
KF = 20
NPH = WCH // KF
AGG_ROWS = 10240
RPT = AGG_ROWS // 16
PAD_DST = N
PK = 4
N4 = AGG_ROWS // PK

@functools.cache
def _sc_mesh():
    return plsc.VectorSubcoreMesh(core_axis_name="c", subcore_axis_name="s")


_SC_PARAMS = pltpu.CompilerParams(use_tc_tiling_on_sc=False)


def _sc_gather(x, srcp, nslots):
    wch = nslots // CHUNK // NW
    nph = wch // KF

    @functools.partial(
        pl.kernel,
        out_type=jax.ShapeDtypeStruct((nslots, LAT), jnp.float32),
        mesh=_sc_mesh(),
        compiler_params=_SC_PARAMS,
        scratch_types=[
            pltpu.VMEM((KF, CHUNK), jnp.int32),
            pltpu.VMEM((KF * CHUNK, LAT), jnp.float32),
            pltpu.SemaphoreType.DMA,
            pltpu.SemaphoreType.DMA,
        ],
    )
    def k(x_hbm, src_hbm, xs_hbm, idx_v, rows_v, semi, semg):
        wid = lax.axis_index("s") * 2 + lax.axis_index("c")
        cb = wid * wch
        for s in range(nph):
            base = (cb + s * KF) * CHUNK
            hs = [
                pltpu.async_copy(
                    src_hbm.at[pl.ds(base + t * CHUNK, CHUNK)], idx_v.at[t], semi
                )
                for t in range(KF)
            ]
            for h in hs:
                h.wait()
            gs = [
                pltpu.async_copy(
                    x_hbm.at[idx_v.at[t]],
                    rows_v.at[pl.ds(t * CHUNK, CHUNK)],
                    semg,
                )
                for t in range(KF)
            ]
            for h in gs:
                h.wait()
            pltpu.sync_copy(rows_v, xs_hbm.at[pl.ds(base, KF * CHUNK)])

    return k(x, srcp)


def _sc_scatter(msgp, dstp, zeros_tile, nslots):
    wch = nslots // CHUNK // NW
    nph = wch // KF

    @functools.partial(
        pl.kernel,
        out_type=jax.ShapeDtypeStruct((2, AGG_ROWS, LAT), jnp.float32),
        mesh=_sc_mesh(),
        compiler_params=_SC_PARAMS,
        scratch_types=[
            pltpu.VMEM((KF, CHUNK), jnp.int32),
            pltpu.VMEM((KF * CHUNK, LAT), jnp.float32),
            pltpu.VMEM_SHARED((AGG_ROWS, LAT), jnp.float32),
            pltpu.SemaphoreType.DMA,
            pltpu.SemaphoreType.DMA,
            pltpu.SemaphoreType.DMA,
        ],
    )
    def k(msg_hbm, dst_hbm, z_hbm, out_hbm, idx_v, msg_v, shared, semi, semm, sems):
        cid = lax.axis_index("c")
        sid = lax.axis_index("s")
        wid = sid * 2 + cid

        if True:
            pltpu.sync_copy(z_hbm, shared.at[pl.ds(sid * RPT, RPT)])
            plsc.subcore_barrier()
            cb = wid * wch
            pend = []
            for s in range(nph):
                base = (cb + s * KF) * CHUNK
                for h in pend:
                    h.wait()
                pend = []
                hs = [
                    pltpu.async_copy(
                        dst_hbm.at[pl.ds(base + t * CHUNK, CHUNK)], idx_v.at[t], semi
                    )
                    for t in range(KF)
                ]
                hm = pltpu.async_copy(
                    msg_hbm.at[pl.ds(base, KF * CHUNK)], msg_v, semm
                )
                for h in hs:
                    h.wait()
                hm.wait()
                pend = [
                    pltpu.async_copy(
                        msg_v.at[pl.ds(t * CHUNK, CHUNK)],
                        shared.at[idx_v.at[t]],
                        sems,
                        add=True,
                    )
                    for t in range(KF)
                ]
            for h in pend:
                h.wait()
            plsc.subcore_barrier()
            pltpu.sync_copy(
                shared.at[pl.ds(sid * RPT, RPT)],
                out_hbm.at[cid, pl.ds(sid * RPT, RPT)],
            )

    return k(msgp, dstp, zeros_tile)


def _sc_deg(dstp, zeros_tile, ones_small):
    @functools.partial(
        pl.kernel,
        out_type=jax.ShapeDtypeStruct((2, AGG_ROWS, LAT), jnp.float32),
        mesh=_sc_mesh(),
        compiler_params=_SC_PARAMS,
        scratch_types=[
            pltpu.VMEM((KF, CHUNK), jnp.int32),
            pltpu.VMEM((KF * CHUNK, LAT), jnp.float32),
            pltpu.VMEM_SHARED((AGG_ROWS, LAT), jnp.float32),
            pltpu.SemaphoreType.DMA,
            pltpu.SemaphoreType.DMA,
        ],
    )
    def k(dst_hbm, z_hbm, ones_hbm, out_hbm, idx_v, msg_v, shared, semi, sems):
        cid = lax.axis_index("c")
        sid = lax.axis_index("s")
        wid = sid * 2 + cid
        pltpu.sync_copy(z_hbm, shared.at[pl.ds(sid * RPT, RPT)])
        pltpu.sync_copy(ones_hbm, msg_v)
        plsc.subcore_barrier()
        cb = wid * WCH
        pend = []
        for s in range(NPH):
            base = (cb + s * KF) * CHUNK
            for h in pend:
                h.wait()
            pend = []
            hs = [
                pltpu.async_copy(
                    dst_hbm.at[pl.ds(base + t * CHUNK, CHUNK)], idx_v.at[t], semi
                )
                for t in range(KF)
            ]
            for h in hs:
                h.wait()
            pend = [
                pltpu.async_copy(
                    msg_v.at[pl.ds(t * CHUNK, CHUNK)],
                    shared.at[idx_v.at[t]],
                    sems,
                    add=True,
                )
                for t in range(KF)
            ]
        for h in pend:
            h.wait()
        plsc.subcore_barrier()
        pltpu.sync_copy(
            shared.at[pl.ds(sid * RPT, RPT)],
            out_hbm.at[cid, pl.ds(sid * RPT, RPT)],
        )

    return k(dstp, zeros_tile, ones_small)


def _edge_msgs(edge_attr, xsp, bp, off):
    ET = 1280
    nslots4 = xsp.shape[0]
    steps = nslots4 * PK // ET
    wk1 = bp["Wk1"]
    bk1 = bp["bk1"].reshape(1, KD)
    wk2 = bp["Wk2"]
    bmat = bp["bk2"].reshape(LAT, LAT).T
    pmat = (
        jnp.arange(LAT * LAT, dtype=jnp.int32)[:, None] // LAT
        == jnp.arange(LAT, dtype=jnp.int32)[None, :]
    ).astype(jnp.float32)

    def body(ea_ref, xs_ref, w1_ref, b1_ref, w2_ref, b_ref, p_ref, msg_ref):
        h = jax.nn.gelu(
            jnp.dot(ea_ref[...], w1_ref[...], preferred_element_type=jnp.float32)
            + b1_ref[...]
        )
        kmat = jnp.dot(h, w2_ref[...], preferred_element_type=jnp.float32)
        xs4 = xs_ref[...]
        xs = jnp.concatenate(
            [xs4[:, LAT * q : LAT * (q + 1)] for q in range(PK)], axis=0
        )
        xsr = jnp.concatenate([xs] * LAT, axis=1)
        msg = jnp.dot(
            kmat * xsr, p_ref[...], preferred_element_type=jnp.float32
        ) + jnp.dot(xs, b_ref[...], preferred_element_type=jnp.float32)
        ET4 = ET // PK
        msg_ref[...] = jnp.concatenate(
            [msg[ET4 * q : ET4 * (q + 1), :] for q in range(PK)], axis=1
        )

    return pl.pallas_call(
        body,
        grid=(steps,),
        in_specs=[
            pl.BlockSpec((ET, EDIM), lambda i: (i + off, 0)),
            pl.BlockSpec((ET // PK, PK * LAT), lambda i: (i, 0)),
            pl.BlockSpec((EDIM, KD), lambda i: (0, 0)),
            pl.BlockSpec((1, KD), lambda i: (0, 0)),
            pl.BlockSpec((KD, LAT * LAT), lambda i: (0, 0)),
            pl.BlockSpec((LAT, LAT), lambda i: (0, 0)),
            pl.BlockSpec((LAT * LAT, LAT), lambda i: (0, 0)),
        ],
        out_specs=pl.BlockSpec((ET // PK, PK * LAT), lambda i: (i, 0)),
        out_shape=jax.ShapeDtypeStruct((nslots4, PK * LAT), jnp.float32),
    )(edge_attr, xsp, wk1, bk1, wk2, bmat, pmat)


def _bdiag(w):
    din, dout = w.shape
    z = jnp.zeros((PK * din, PK * dout), jnp.float32)
    for i in range(PK):
        z = z.at[i * din : (i + 1) * din, i * dout : (i + 1) * dout].set(w)
    return z


def _tile_b(b):
    return jnp.concatenate([b.reshape(1, -1)] * PK, axis=1)


def _encode(xin4, pp):
    def body(xi_ref, w1_ref, b1_ref, w2_ref, b2_ref, x0_ref):
        h = jax.nn.gelu(
            jnp.dot(xi_ref[...], w1_ref[...], preferred_element_type=jnp.float32)
            + b1_ref[...]
        )
        x0_ref[...] = jax.nn.gelu(
            jnp.dot(h, w2_ref[...], preferred_element_type=jnp.float32) + b2_ref[...]
        )

    return pl.pallas_call(
        body,
        out_shape=jax.ShapeDtypeStruct((N4, PK * LAT), jnp.float32),
    )(
        xin4,
        _bdiag(pp["W1"]),
        _tile_b(pp["b1"]),
        _bdiag(pp["W2"]),
        _tile_b(pp["b2"]),
    )


def _agg4(pa_ref, pb_ref, dp_ref):
    deg = dp_ref[0] + dp_ref[1]
    rd = 1.0 / jnp.maximum(deg, 1.0)
    return (pa_ref[0] + pa_ref[1] + pb_ref[0] + pb_ref[1]) * rd


def _update(x4, pa4, pb4, dparts4, wrd, brt, apply_gelu):
    def body(x_ref, pa_ref, pb_ref, dp_ref, wr_ref, br_ref, o_ref):
        y = (
            jnp.dot(x_ref[...], wr_ref[...], preferred_element_type=jnp.float32)
            + br_ref[...]
            + _agg4(pa_ref, pb_ref, dp_ref)
        )
        if apply_gelu:
            y = jax.nn.gelu(y)
        o_ref[...] = y

    return pl.pallas_call(
        body,
        out_shape=jax.ShapeDtypeStruct((N4, PK * LAT), jnp.float32),
    )(x4, pa4, pb4, dparts4, wrd, brt)


def _update_decode(x4, pa4, pb4, dparts4, wrd, brt, dp):
    w2s = _bdiag(dp["W2"])
    b2t = jnp.concatenate([dp["b2"].reshape(1, 1)] * PK, axis=1)

    def body(x_ref, pa_ref, pb_ref, dp_ref, wr_ref, br_ref, w1_ref, b1_ref,
             w2s_ref, b2_ref, o_ref):
        y = (
            jnp.dot(x_ref[...], wr_ref[...], preferred_element_type=jnp.float32)
            + br_ref[...]
            + _agg4(pa_ref, pb_ref, dp_ref)
        )
        h = jax.nn.gelu(
            jnp.dot(y, w1_ref[...], preferred_element_type=jnp.float32)
            + b1_ref[...]
        )
        o_ref[...] = (
            jnp.dot(h, w2s_ref[...], preferred_element_type=jnp.float32)
            + b2_ref[...]
        )

    return pl.pallas_call(
        body,
        out_shape=jax.ShapeDtypeStruct((N4, PK), jnp.float32),
    )(
        x4,
        pa4,
        pb4,
        dparts4,
        wrd,
        brt,
        _bdiag(dp["W1"]),
        _tile_b(dp["b1"]),
        w2s,
        b2t,
    )


def kernel(nodes, grid, edge_index, edge_attr, batch_size, image_size, params):
    src = edge_index[0].astype(jnp.int32)
    dst = edge_index[1].astype(jnp.int32)
    srcp = jnp.concatenate([src, jnp.zeros((EPAD - E,), jnp.int32)])
    dstp = jnp.concatenate([dst, jnp.full((EPAD - E,), PAD_DST, jnp.int32)])
    s_ids = np.arange(EPAD)
    blk, t = s_ids // 1280, s_ids % 1280
    jg = jnp.asarray(blk * 1280 + (t % PK) * (1280 // PK) + t // PK)
    srcp = srcp[jg]
    dstp = dstp[jg]
    zeros_tile = jnp.zeros((RPT, LAT), jnp.float32)
    ones_small = jnp.ones((KF * CHUNK, LAT), jnp.float32)

    EH = EPAD // 2
    srcpA, srcpB = srcp[:EH], srcp[EH:]
    dstpA, dstpB = dstp[:EH], dstp[EH:]
    eap = jnp.pad(edge_attr, ((0, EPAD - E), (0, 0)))

    dparts = _sc_deg(dstp, zeros_tile, ones_small)
    dparts4 = dparts.reshape(2, N4, PK * LAT)
    xin = jnp.concatenate([nodes, grid], axis=1)
    xin4 = jnp.pad(xin, ((0, AGG_ROWS - N), (0, 0))).reshape(N4, PK * 12)
    x4 = _encode(xin4, params["projector"])

    nblk = EH // 1280
    for b in range(2):
        bp = params["blocks"][b]
        wrd = _bdiag(bp["Wr"])
        brt = _tile_b(bp["br"])
        for p in range(2):
            xf = x4.reshape(AGG_ROWS, LAT)
            xsA = _sc_gather(xf, srcpA, EH).reshape(EH // PK, PK * LAT)
            xsB = _sc_gather(xf, srcpB, EH).reshape(EH // PK, PK * LAT)
            msgA = _edge_msgs(eap, xsA, bp, 0)
            msgB = _edge_msgs(eap, xsB, bp, nblk)
            pA = _sc_scatter(msgA.reshape(EH, LAT), dstpA, zeros_tile, EH)
            pB = _sc_scatter(msgB.reshape(EH, LAT), dstpB, zeros_tile, EH)
            pA4 = pA.reshape(2, N4, PK * LAT)
            pB4 = pB.reshape(2, N4, PK * LAT)
            if b == 1 and p == 1:
                out4 = _update_decode(
                    x4, pA4, pB4, dparts4, wrd, brt, params["decoder"]
                )
                return out4.reshape(AGG_ROWS, 1)[:N]
            x4 = _update(x4, pA4, pB4, dparts4, wrd, brt, True)


# speedup vs baseline: 4.4452x; 1.0000x over previous
import functools

import jax
import jax.numpy as jnp
import numpy as np
from jax import lax
from jax.experimental import pallas as pl
from jax.experimental.pallas import tpu as pltpu
from jax.experimental.pallas import tpu_sc as plsc

N = 10000
E = 160000
LAT = 32
KD = 64
EDIM = 5

CHUNK = 128
EPAD = 163840
NW = 32
WCH = EPAD // CHUNK // NW
KF = 20
NPH = WCH // KF
AGG_ROWS = 10240
RPT = AGG_ROWS // 16
PAD_DST = N
PK = 4
N4 = AGG_ROWS // PK

@functools.cache
def _sc_mesh():
    return plsc.VectorSubcoreMesh(core_axis_name="c", subcore_axis_name="s")


_SC_PARAMS = pltpu.CompilerParams(use_tc_tiling_on_sc=False)


def _sc_gather(x, srcp, nslots):
    wch = nslots // CHUNK // NW
    nph = wch // KF

    @functools.partial(
        pl.kernel,
        out_type=jax.ShapeDtypeStruct((nslots, LAT), jnp.float32),
        mesh=_sc_mesh(),
        compiler_params=_SC_PARAMS,
        scratch_types=[
            pltpu.VMEM((KF, CHUNK), jnp.int32),
            pltpu.VMEM((KF * CHUNK, LAT), jnp.float32),
            pltpu.SemaphoreType.DMA,
            pltpu.SemaphoreType.DMA,
        ],
    )
    def k(x_hbm, src_hbm, xs_hbm, idx_v, rows_v, semi, semg):
        wid = lax.axis_index("s") * 2 + lax.axis_index("c")
        cb = wid * wch
        for s in range(nph):
            base = (cb + s * KF) * CHUNK
            hs = [
                pltpu.async_copy(
                    src_hbm.at[pl.ds(base + t * CHUNK, CHUNK)], idx_v.at[t], semi
                )
                for t in range(KF)
            ]
            for h in hs:
                h.wait()
            gs = [
                pltpu.async_copy(
                    x_hbm.at[idx_v.at[t]],
                    rows_v.at[pl.ds(t * CHUNK, CHUNK)],
                    semg,
                )
                for t in range(KF)
            ]
            for h in gs:
                h.wait()
            pltpu.sync_copy(rows_v, xs_hbm.at[pl.ds(base, KF * CHUNK)])

    return k(x, srcp)


def _sc_scatter(msgp, dstp, zeros_tile, nslots):
    wch = nslots // CHUNK // NW
    nph = wch // KF

    @functools.partial(
        pl.kernel,
        out_type=jax.ShapeDtypeStruct((2, AGG_ROWS, LAT), jnp.float32),
        mesh=_sc_mesh(),
        compiler_params=_SC_PARAMS,
        scratch_types=[
            pltpu.VMEM((KF, CHUNK), jnp.int32),
            pltpu.VMEM((KF * CHUNK, LAT), jnp.float32),
            pltpu.VMEM_SHARED((AGG_ROWS, LAT), jnp.float32),
            pltpu.SemaphoreType.DMA,
            pltpu.SemaphoreType.DMA,
            pltpu.SemaphoreType.DMA,
        ],
    )
    def k(msg_hbm, dst_hbm, z_hbm, out_hbm, idx_v, msg_v, shared, semi, semm, sems):
        cid = lax.axis_index("c")
        sid = lax.axis_index("s")
        wid = sid * 2 + cid
        pltpu.sync_copy(z_hbm, shared.at[pl.ds(sid * RPT, RPT)])
        plsc.subcore_barrier()
        cb = wid * wch
        pend = []
        for s in range(nph):
            base = (cb + s * KF) * CHUNK
            for h in pend:
                h.wait()
            pend = []
            hs = [
                pltpu.async_copy(
                    dst_hbm.at[pl.ds(base + t * CHUNK, CHUNK)], idx_v.at[t], semi
                )
                for t in range(KF)
            ]
            hm = pltpu.async_copy(msg_hbm.at[pl.ds(base, KF * CHUNK)], msg_v, semm)
            for h in hs:
                h.wait()
            hm.wait()
            pend = [
                pltpu.async_copy(
                    msg_v.at[pl.ds(t * CHUNK, CHUNK)],
                    shared.at[idx_v.at[t]],
                    sems,
                    add=True,
                )
                for t in range(KF)
            ]
        for h in pend:
            h.wait()
        plsc.subcore_barrier()
        pltpu.sync_copy(
            shared.at[pl.ds(sid * RPT, RPT)],
            out_hbm.at[cid, pl.ds(sid * RPT, RPT)],
        )

    return k(msgp, dstp, zeros_tile)


def _sc_deg(dstp, zeros_tile, ones_small):
    @functools.partial(
        pl.kernel,
        out_type=jax.ShapeDtypeStruct((2, AGG_ROWS, LAT), jnp.float32),
        mesh=_sc_mesh(),
        compiler_params=_SC_PARAMS,
        scratch_types=[
            pltpu.VMEM((KF, CHUNK), jnp.int32),
            pltpu.VMEM((KF * CHUNK, LAT), jnp.float32),
            pltpu.VMEM_SHARED((AGG_ROWS, LAT), jnp.float32),
            pltpu.SemaphoreType.DMA,
            pltpu.SemaphoreType.DMA,
        ],
    )
    def k(dst_hbm, z_hbm, ones_hbm, out_hbm, idx_v, msg_v, shared, semi, sems):
        cid = lax.axis_index("c")
        sid = lax.axis_index("s")
        wid = sid * 2 + cid
        pltpu.sync_copy(z_hbm, shared.at[pl.ds(sid * RPT, RPT)])
        pltpu.sync_copy(ones_hbm, msg_v)
        plsc.subcore_barrier()
        cb = wid * WCH
        pend = []
        for s in range(NPH):
            base = (cb + s * KF) * CHUNK
            for h in pend:
                h.wait()
            pend = []
            hs = [
                pltpu.async_copy(
                    dst_hbm.at[pl.ds(base + t * CHUNK, CHUNK)], idx_v.at[t], semi
                )
                for t in range(KF)
            ]
            for h in hs:
                h.wait()
            pend = [
                pltpu.async_copy(
                    msg_v.at[pl.ds(t * CHUNK, CHUNK)],
                    shared.at[idx_v.at[t]],
                    sems,
                    add=True,
                )
                for t in range(KF)
            ]
        for h in pend:
            h.wait()
        plsc.subcore_barrier()
        pltpu.sync_copy(
            shared.at[pl.ds(sid * RPT, RPT)],
            out_hbm.at[cid, pl.ds(sid * RPT, RPT)],
        )

    return k(dstp, zeros_tile, ones_small)


def _edge_msgs(edge_attr, xsp, bp, off):
    ET = 1280
    nslots4 = xsp.shape[0]
    steps = nslots4 * PK // ET
    wk1 = bp["Wk1"]
    bk1 = bp["bk1"].reshape(1, KD)
    wk2 = bp["Wk2"]
    bmat = bp["bk2"].reshape(LAT, LAT).T
    pmat = (
        jnp.arange(LAT * LAT, dtype=jnp.int32)[:, None] // LAT
        == jnp.arange(LAT, dtype=jnp.int32)[None, :]
    ).astype(jnp.float32)

    def body(ea_ref, xs_ref, w1_ref, b1_ref, w2_ref, b_ref, p_ref, msg_ref):
        h = jax.nn.gelu(
            jnp.dot(ea_ref[...], w1_ref[...], preferred_element_type=jnp.float32)
            + b1_ref[...]
        )
        kmat = jnp.dot(h, w2_ref[...], preferred_element_type=jnp.float32)
        xs4 = xs_ref[...]
        xs = jnp.concatenate(
            [xs4[:, LAT * q : LAT * (q + 1)] for q in range(PK)], axis=0
        )
        xsr = jnp.concatenate([xs] * LAT, axis=1)
        msg = jnp.dot(
            kmat * xsr, p_ref[...], preferred_element_type=jnp.float32
        ) + jnp.dot(xs, b_ref[...], preferred_element_type=jnp.float32)
        ET4 = ET // PK
        msg_ref[...] = jnp.concatenate(
            [msg[ET4 * q : ET4 * (q + 1), :] for q in range(PK)], axis=1
        )

    return pl.pallas_call(
        body,
        grid=(steps,),
        in_specs=[
            pl.BlockSpec((ET, EDIM), lambda i: (i + off, 0)),
            pl.BlockSpec((ET // PK, PK * LAT), lambda i: (i, 0)),
            pl.BlockSpec((EDIM, KD), lambda i: (0, 0)),
            pl.BlockSpec((1, KD), lambda i: (0, 0)),
            pl.BlockSpec((KD, LAT * LAT), lambda i: (0, 0)),
            pl.BlockSpec((LAT, LAT), lambda i: (0, 0)),
            pl.BlockSpec((LAT * LAT, LAT), lambda i: (0, 0)),
        ],
        out_specs=pl.BlockSpec((ET // PK, PK * LAT), lambda i: (i, 0)),
        out_shape=jax.ShapeDtypeStruct((nslots4, PK * LAT), jnp.float32),
    )(edge_attr, xsp, wk1, bk1, wk2, bmat, pmat)


def _bdiag(w):
    din, dout = w.shape
    z = jnp.zeros((PK * din, PK * dout), jnp.float32)
    for i in range(PK):
        z = z.at[i * din : (i + 1) * din, i * dout : (i + 1) * dout].set(w)
    return z


def _tile_b(b):
    return jnp.concatenate([b.reshape(1, -1)] * PK, axis=1)


def _encode(xin4, pp):
    def body(xi_ref, w1_ref, b1_ref, w2_ref, b2_ref, x0_ref):
        h = jax.nn.gelu(
            jnp.dot(xi_ref[...], w1_ref[...], preferred_element_type=jnp.float32)
            + b1_ref[...]
        )
        x0_ref[...] = jax.nn.gelu(
            jnp.dot(h, w2_ref[...], preferred_element_type=jnp.float32) + b2_ref[...]
        )

    return pl.pallas_call(
        body,
        out_shape=jax.ShapeDtypeStruct((N4, PK * LAT), jnp.float32),
    )(
        xin4,
        _bdiag(pp["W1"]),
        _tile_b(pp["b1"]),
        _bdiag(pp["W2"]),
        _tile_b(pp["b2"]),
    )


def _agg4(pa_ref, pb_ref, dp_ref):
    deg = dp_ref[0] + dp_ref[1]
    rd = 1.0 / jnp.maximum(deg, 1.0)
    return (pa_ref[0] + pa_ref[1] + pb_ref[0] + pb_ref[1]) * rd


def _update(x4, pa4, pb4, dparts4, wrd, brt, apply_gelu):
    def body(x_ref, pa_ref, pb_ref, dp_ref, wr_ref, br_ref, o_ref):
        y = (
            jnp.dot(x_ref[...], wr_ref[...], preferred_element_type=jnp.float32)
            + br_ref[...]
            + _agg4(pa_ref, pb_ref, dp_ref)
        )
        if apply_gelu:
            y = jax.nn.gelu(y)
        o_ref[...] = y

    return pl.pallas_call(
        body,
        out_shape=jax.ShapeDtypeStruct((N4, PK * LAT), jnp.float32),
    )(x4, pa4, pb4, dparts4, wrd, brt)


def _update_decode(x4, pa4, pb4, dparts4, wrd, brt, dp):
    w2s = _bdiag(dp["W2"])
    b2t = jnp.concatenate([dp["b2"].reshape(1, 1)] * PK, axis=1)

    def body(x_ref, pa_ref, pb_ref, dp_ref, wr_ref, br_ref, w1_ref, b1_ref,
             w2s_ref, b2_ref, o_ref):
        y = (
            jnp.dot(x_ref[...], wr_ref[...], preferred_element_type=jnp.float32)
            + br_ref[...]
            + _agg4(pa_ref, pb_ref, dp_ref)
        )
        h = jax.nn.gelu(
            jnp.dot(y, w1_ref[...], preferred_element_type=jnp.float32)
            + b1_ref[...]
        )
        o_ref[...] = (
            jnp.dot(h, w2s_ref[...], preferred_element_type=jnp.float32)
            + b2_ref[...]
        )

    return pl.pallas_call(
        body,
        out_shape=jax.ShapeDtypeStruct((N4, PK), jnp.float32),
    )(
        x4,
        pa4,
        pb4,
        dparts4,
        wrd,
        brt,
        _bdiag(dp["W1"]),
        _tile_b(dp["b1"]),
        w2s,
        b2t,
    )


def kernel(nodes, grid, edge_index, edge_attr, batch_size, image_size, params):
    src = edge_index[0].astype(jnp.int32)
    dst = edge_index[1].astype(jnp.int32)
    srcp = jnp.concatenate([src, jnp.zeros((EPAD - E,), jnp.int32)])
    dstp = jnp.concatenate([dst, jnp.full((EPAD - E,), PAD_DST, jnp.int32)])
    s_ids = np.arange(EPAD)
    blk, t = s_ids // 1280, s_ids % 1280
    jg = jnp.asarray(blk * 1280 + (t % PK) * (1280 // PK) + t // PK)
    srcp = srcp[jg]
    dstp = dstp[jg]
    zeros_tile = jnp.zeros((RPT, LAT), jnp.float32)
    ones_small = jnp.ones((KF * CHUNK, LAT), jnp.float32)

    EH = EPAD // 2
    srcpA, srcpB = srcp[:EH], srcp[EH:]
    dstpA, dstpB = dstp[:EH], dstp[EH:]
    eap = jnp.pad(edge_attr, ((0, EPAD - E), (0, 0)))

    dparts = _sc_deg(dstp, zeros_tile, ones_small)
    dparts4 = dparts.reshape(2, N4, PK * LAT)
    xin = jnp.concatenate([nodes, grid], axis=1)
    xin4 = jnp.pad(xin, ((0, AGG_ROWS - N), (0, 0))).reshape(N4, PK * 12)
    x4 = _encode(xin4, params["projector"])

    nblk = EH // 1280
    for b in range(2):
        bp = params["blocks"][b]
        wrd = _bdiag(bp["Wr"])
        brt = _tile_b(bp["br"])
        for p in range(2):
            xf = x4.reshape(AGG_ROWS, LAT)
            xsA = _sc_gather(xf, srcpA, EH).reshape(EH // PK, PK * LAT)
            xsB = _sc_gather(xf, srcpB, EH).reshape(EH // PK, PK * LAT)
            msgA = _edge_msgs(eap, xsA, bp, 0)
            msgB = _edge_msgs(eap, xsB, bp, nblk)
            pA = _sc_scatter(msgA.reshape(EH, LAT), dstpA, zeros_tile, EH)
            pB = _sc_scatter(msgB.reshape(EH, LAT), dstpB, zeros_tile, EH)
            pA4 = pA.reshape(2, N4, PK * LAT)
            pB4 = pB.reshape(2, N4, PK * LAT)
            if b == 1 and p == 1:
                out4 = _update_decode(
                    x4, pA4, pB4, dparts4, wrd, brt, params["decoder"]
                )
                return out4.reshape(AGG_ROWS, 1)[:N]
            x4 = _update(x4, pA4, pB4, dparts4, wrd, brt, True)
